# baseline (speedup 1.0000x reference)
import functools

import jax
import jax.numpy as jnp
from jax import lax
from jax.experimental import pallas as pl
from jax.experimental.pallas import tpu as pltpu
from jax.experimental.pallas import tpu_sc as plsc

_B = 16384 * 20
_D = 64
_NC = 2
_NS = 16
_NW = _NC * _NS
_BPW = _B // _NW
_CHUNK = 512
_NCHUNK = _BPW // _CHUNK
_NBUF = 2

_mesh = plsc.VectorSubcoreMesh(core_axis_name="c", subcore_axis_name="s")


@functools.partial(
    pl.kernel,
    out_type=jax.ShapeDtypeStruct((_B, _D), jnp.float32),
    mesh=_mesh,
    scratch_types=[
        pltpu.VMEM((_BPW,), jnp.int32),
        pltpu.VMEM((_NBUF, _CHUNK, _D), jnp.float32),
        pltpu.SemaphoreType.DMA,
        pltpu.SemaphoreType.DMA,
        pltpu.SemaphoreType.DMA,
    ],
    compiler_params=pltpu.CompilerParams(use_tc_tiling_on_sc=False),
)
def _gather_kernel(idx_hbm, table_hbm, out_hbm, idx_v, rows_v, gsem, osem0, osem1):
    wid = lax.axis_index("s") * _NC + lax.axis_index("c")
    base = wid * _BPW
    pltpu.sync_copy(idx_hbm.at[pl.ds(base, _BPW)], idx_v)

    osems = (osem0, osem1)

    def gather_start(g, slot):
        return pltpu.async_copy(
            table_hbm.at[idx_v.at[pl.ds(g * _CHUNK, _CHUNK)]],
            rows_v.at[slot],
            gsem,
        )

    def out_start(g, slot):
        return pltpu.async_copy(
            rows_v.at[slot],
            out_hbm.at[pl.ds(base + g * _CHUNK, _CHUNK)],
            osems[slot],
        )

    handles = [None] * _NBUF
    for g in range(_NCHUNK):
        slot = g % _NBUF
        if handles[slot] is not None:
            handles[slot].wait()
        gather_start(g, slot).wait()
        handles[slot] = out_start(g, slot)
    for h in handles:
        if h is not None:
            h.wait()


def kernel(indices, weight):
    idx_flat = indices.reshape(-1).astype(jnp.int32)
    out = _gather_kernel(idx_flat, weight)
    return out.reshape(indices.shape + (weight.shape[-1],))
